# baseline (speedup 1.0000x reference)
import functools

import jax
import jax.numpy as jnp
from jax import lax
from jax.experimental import pallas as pl
from jax.experimental.pallas import tpu as pltpu
from jax.experimental.pallas import tpu_sc as plsc

_B = 4
_N = 4096
_NC = 2
_NS = 16
_NW = _NC * _NS
_WPB = _NW // _B
_CHUNK = _N // _WPB
_L = 16
_G = 2
_QB = _CHUNK // (_L * _G)

_NEG = -3.4e38


def _sc_chamfer(pos_t, xhat_t, pos_h, xhat_h):
    mesh = plsc.VectorSubcoreMesh(core_axis_name="c", subcore_axis_name="s")

    @functools.partial(
        pl.kernel,
        mesh=mesh,
        out_type=jax.ShapeDtypeStruct((_NW, _L), jnp.float32),
        scratch_types=[
            pltpu.VMEM((3, _N), jnp.float32),
            pltpu.VMEM((3, _N), jnp.float32),
            pltpu.VMEM((_N,), jnp.float32),
            pltpu.VMEM((_N,), jnp.float32),
            pltpu.VMEM((_L,), jnp.float32),
        ],
    )
    def k(pos_hbm, xhat_hbm, ph_hbm, xh_hbm, out_hbm,
          a_ref, b_ref, ah_ref, bh_ref, o_ref):
        wid = lax.axis_index("s") * _NC + lax.axis_index("c")
        bat = wid // _WPB
        chk = wid % _WPB
        pltpu.sync_copy(pos_hbm.at[bat], a_ref)
        pltpu.sync_copy(xhat_hbm.at[bat], b_ref)
        pltpu.sync_copy(ph_hbm.at[bat], ah_ref)
        pltpu.sync_copy(xh_hbm.at[bat], bh_ref)

        def one_direction(q_ref, qh_ref, s_ref, sh_ref, acc0):
            def qblock(gb, acc):
                qoff = chk * _CHUNK + gb * (_L * _G)
                qx = [q_ref[0, pl.ds(qoff + i * _L, _L)] for i in range(_G)]
                qy = [q_ref[1, pl.ds(qoff + i * _L, _L)] for i in range(_G)]
                qz = [q_ref[2, pl.ds(qoff + i * _L, _L)] for i in range(_G)]
                qh = [qh_ref[pl.ds(qoff + i * _L, _L)] for i in range(_G)]

                def jloop(j, ms):
                    soff = j * _L
                    sxv = s_ref[0, pl.ds(soff, _L)]
                    syv = s_ref[1, pl.ds(soff, _L)]
                    szv = s_ref[2, pl.ds(soff, _L)]
                    shv = sh_ref[pl.ds(soff, _L)]
                    ms = list(ms)
                    for e in range(_L):
                        sx = sxv[e]
                        sy = syv[e]
                        sz = szv[e]
                        sh = shv[e]
                        for i in range(_G):
                            t = qx[i] * sx + qy[i] * sy + qz[i] * sz
                            ms[i] = jnp.maximum(ms[i], t - sh)
                    return tuple(ms)

                ms = lax.fori_loop(
                    0, _N // _L, jloop,
                    tuple(jnp.full((_L,), _NEG, jnp.float32)
                          for _ in range(_G)))
                for i in range(_G):
                    acc = acc + (qh[i] - ms[i])
                return acc

            return lax.fori_loop(0, _QB, qblock, acc0)

        s = one_direction(a_ref, ah_ref, b_ref, bh_ref,
                          jnp.zeros((_L,), jnp.float32))
        s = one_direction(b_ref, bh_ref, a_ref, ah_ref, s)
        o_ref[...] = s + s
        pltpu.sync_copy(o_ref, out_hbm.at[wid])

    return k(pos_t, xhat_t, pos_h, xhat_h)


def kernel(pos, x_hat):
    pos_t = jnp.transpose(pos, (0, 2, 1))
    xhat_t = jnp.transpose(x_hat, (0, 2, 1))
    pos_h = 0.5 * jnp.sum(pos * pos, axis=-1)
    xhat_h = 0.5 * jnp.sum(x_hat * x_hat, axis=-1)
    partial = _sc_chamfer(pos_t, xhat_t, pos_h, xhat_h)
    return jnp.sum(partial) * jnp.float32(1.0 / (_B * _N))


# speedup vs baseline: 8.7477x; 8.7477x over previous
import functools

import jax
import jax.numpy as jnp
from jax import lax
from jax.experimental import pallas as pl
from jax.experimental.pallas import tpu as pltpu
from jax.experimental.pallas import tpu_sc as plsc

_B = 4
_N = 4096
_NC = 2
_NS = 16
_NW = _NC * _NS
_WPB = _NW // _B
_CHUNK = _N // _WPB
_L = 16
_G = 2
_QB = _CHUNK // (_L * _G)

_NEG = -3.4e38


def _sc_chamfer(pos_t, xhat_t, pos_h, xhat_h):
    mesh = plsc.VectorSubcoreMesh(core_axis_name="c", subcore_axis_name="s")

    @functools.partial(
        pl.kernel,
        mesh=mesh,
        out_type=jax.ShapeDtypeStruct((_NW, _L), jnp.float32),
        scratch_types=[
            pltpu.VMEM((3, _N), jnp.float32),
            pltpu.VMEM((3, _N), jnp.float32),
            pltpu.VMEM((_N,), jnp.float32),
            pltpu.VMEM((_N,), jnp.float32),
            pltpu.VMEM((_L,), jnp.float32),
        ],
    )
    def k(pos_hbm, xhat_hbm, ph_hbm, xh_hbm, out_hbm,
          a_ref, b_ref, ah_ref, bh_ref, o_ref):
        wid = lax.axis_index("s") * _NC + lax.axis_index("c")
        bat = wid // _WPB
        chk = wid % _WPB
        pltpu.sync_copy(pos_hbm.at[bat], a_ref)
        pltpu.sync_copy(xhat_hbm.at[bat], b_ref)
        pltpu.sync_copy(ph_hbm.at[bat], ah_ref)
        pltpu.sync_copy(xh_hbm.at[bat], bh_ref)

        def one_direction(q_ref, qh_ref, s_ref, sh_ref, acc0):
            def qblock(gb, acc):
                qoff = chk * _CHUNK + gb * (_L * _G)
                qx = [q_ref[0, pl.ds(qoff + i * _L, _L)] for i in range(_G)]
                qy = [q_ref[1, pl.ds(qoff + i * _L, _L)] for i in range(_G)]
                qz = [q_ref[2, pl.ds(qoff + i * _L, _L)] for i in range(_G)]
                qh = [qh_ref[pl.ds(qoff + i * _L, _L)] for i in range(_G)]

                def jloop(j, ms):
                    soff = j * _L
                    sxv = s_ref[0, pl.ds(soff, _L)]
                    syv = s_ref[1, pl.ds(soff, _L)]
                    szv = s_ref[2, pl.ds(soff, _L)]
                    shv = sh_ref[pl.ds(soff, _L)]
                    ms = list(ms)
                    for e in range(_L):
                        sx = sxv[e]
                        sy = syv[e]
                        sz = szv[e]
                        sh = shv[e]
                        for i in range(_G):
                            t = qx[i] * sx + qy[i] * sy + qz[i] * sz
                            ms[i] = jnp.maximum(ms[i], t - sh)
                    return tuple(ms)

                ms = lax.fori_loop(
                    0, _N // _L, jloop,
                    tuple(jnp.full((_L,), _NEG, jnp.float32)
                          for _ in range(_G)))
                for i in range(_G):
                    acc = acc + (qh[i] - ms[i])
                return acc

            return lax.fori_loop(0, _QB, qblock, acc0)

        s = one_direction(a_ref, ah_ref, b_ref, bh_ref,
                          jnp.zeros((_L,), jnp.float32))
        s = one_direction(b_ref, bh_ref, a_ref, ah_ref, s)
        o_ref[...] = s + s
        pltpu.sync_copy(o_ref, out_hbm.at[wid])

    return k(pos_t, xhat_t, pos_h, xhat_h)


_TCQ = 256
_TCKB = _N // _TCQ


def _tc_chamfer(q6, s6, nb):
    def body(q_ref, s_ref, out_ref, cmax_ref):
        b = pl.program_id(0)
        k = pl.program_id(1)

        @pl.when(jnp.logical_and(b == 0, k == 0))
        def _():
            out_ref[0, 0] = jnp.float32(0.0)

        @pl.when(k == 0)
        def _():
            cmax_ref[...] = jnp.full((8, _N), _NEG, jnp.float32)

        m = jnp.dot(q_ref[0], s_ref[0],
                    preferred_element_type=jnp.float32)
        out_ref[0, 0] += jnp.sum(jnp.max(m, axis=1))
        c = cmax_ref[...]
        for i in range(_TCQ // 8):
            c = jnp.maximum(c, m[i * 8:(i + 1) * 8, :])
        cmax_ref[...] = c

        @pl.when(k == _TCKB - 1)
        def _():
            out_ref[0, 0] += jnp.sum(jnp.max(cmax_ref[...], axis=0))

    return pl.pallas_call(
        body,
        grid=(nb, _TCKB),
        in_specs=[
            pl.BlockSpec((1, _TCQ, 8), lambda b, k: (b, k, 0)),
            pl.BlockSpec((1, 8, _N), lambda b, k: (b, 0, 0)),
        ],
        out_specs=pl.BlockSpec(memory_space=pltpu.SMEM),
        out_shape=jax.ShapeDtypeStruct((1, 1), jnp.float32),
        scratch_shapes=[pltpu.VMEM((8, _N), jnp.float32)],
    )(q6, s6)


def kernel(pos, x_hat):
    pos_h = 0.5 * jnp.sum(pos * pos, axis=-1)
    xhat_h = 0.5 * jnp.sum(x_hat * x_hat, axis=-1)
    ones = jnp.ones((_B, _N, 1), jnp.float32)
    zeros = jnp.zeros((_B, _N, 3), jnp.float32)
    q6 = jnp.concatenate(
        [pos, ones, -pos_h[..., None], zeros], axis=-1)
    s6 = jnp.concatenate(
        [x_hat, -xhat_h[..., None], ones, zeros], axis=-1)
    s6 = jnp.transpose(s6, (0, 2, 1))
    msum = _tc_chamfer(q6, s6, _B)
    return -2.0 * msum[0, 0] * jnp.float32(1.0 / (_B * _N))
